# baseline (speedup 1.0000x reference)
import functools

import jax
import jax.numpy as jnp
from jax.experimental import pallas as pl
from jax.experimental.pallas import tpu as pltpu


B, D, H, R = 16384, 2048, 128, 16
TB = 1024
NT = B // TB


def _router_kernel(x_ref, w1_ref, b1_ref, w2_ref, b2_ref, rb_ref,
                   sel_ref, prob_ref):
    i = pl.program_id(0)
    rows = pl.ds(i * TB, TB)
    h = jnp.maximum(
        jnp.dot(x_ref[...], w1_ref[...], preferred_element_type=jnp.float32)
        + b1_ref[...], 0.0)
    logits = (jnp.dot(h, w2_ref[...], preferred_element_type=jnp.float32)
              + b2_ref[...] + rb_ref[...])
    m = jnp.max(logits, axis=-1, keepdims=True)
    e = jnp.exp(logits - m)
    prob_ref[rows, :] = e * (1.0 / jnp.sum(e, axis=-1, keepdims=True))
    lane = jax.lax.broadcasted_iota(jnp.int32, logits.shape, 1)
    sel_ref[rows, :] = jnp.min(jnp.where(logits == m, lane, R), axis=-1,
                               keepdims=True)


@functools.partial(jax.jit, static_argnames=())
def kernel(x, W1, b1, W2, b2, route_bias):
    sel, probs = pl.pallas_call(
        _router_kernel,
        grid=(NT,),
        in_specs=[
            pl.BlockSpec((TB, D), lambda i: (i, 0)),
            pl.BlockSpec((D, H), lambda i: (0, 0)),
            pl.BlockSpec((1, H), lambda i: (0, 0)),
            pl.BlockSpec((H, R), lambda i: (0, 0)),
            pl.BlockSpec((1, R), lambda i: (0, 0)),
            pl.BlockSpec((1, R), lambda i: (0, 0)),
        ],
        out_specs=[
            pl.BlockSpec((B, 1), lambda i: (0, 0)),
            pl.BlockSpec((B, R), lambda i: (0, 0)),
        ],
        out_shape=[
            jax.ShapeDtypeStruct((B, 1), jnp.int32),
            jax.ShapeDtypeStruct((B, R), jnp.float32),
        ],
        compiler_params=pltpu.CompilerParams(
            dimension_semantics=("arbitrary",)),
    )(x, W1, b1.reshape(1, H), W2, b2.reshape(1, R),
      route_bias.reshape(1, R))
    return (sel.reshape(B), probs)
